# baseline (speedup 1.0000x reference)
import jax
import jax.numpy as jnp
from jax import lax
from jax.experimental import pallas as pl
from jax.experimental.pallas import tpu as pltpu
from jax.experimental.pallas import tpu_sc as plsc

N_NODES = 10000
N_EDGES = 160000
H = 256
HH = 128
ED = 16

BN = 1000
BE = 2000

NC = 2
NS = 16
L = 16

EPT1 = N_EDGES // NS
G1 = EPT1 // L
RBG = 25
RB = RBG * L
NA0 = 5008
NA1 = EPT1 - NA0
G0 = NA0 // L
DR = 640
TB = 8
TROWS = TB * L
TC_ = ED + L
TR = 10240

NW = NC * NS
EPT2 = N_EDGES // NS
BCH = 80
NCH = EPT2 // BCH
NBM = 3
NBI = 5
RPT = N_NODES // NS


def _node_kernel(src_x, dst_x, w_src, b_src, w_dst, b_dst, w_msg, w1, w2,
                 w_edge, b_edge, b_msg, w_out,
                 a_src_o, a_dst_o, m01_o, wem2_o, bem2_o):
    xs = jnp.dot(src_x[...], w_src[...], preferred_element_type=jnp.float32) + b_src[...]
    a_src_o[...] = jnp.dot(jnp.tanh(xs), w1[...], preferred_element_type=jnp.float32)
    m = jnp.dot(xs, w_msg[...], preferred_element_type=jnp.float32)
    m01_o[0] = m[:, :HH]
    m01_o[1] = m[:, HH:]
    xd = jnp.dot(dst_x[...], w_dst[...], preferred_element_type=jnp.float32) + b_dst[...]
    a_dst_o[...] = jnp.dot(jnp.tanh(xd), w2[...], preferred_element_type=jnp.float32)

    @pl.when(pl.program_id(0) == 0)
    def _():
        wo_b = w_out[H:, :]
        wem = jnp.dot(w_edge[...], w_msg[...], preferred_element_type=jnp.float32)
        wem2_o[...] = jnp.dot(wem, wo_b, preferred_element_type=jnp.float32)
        bem = jnp.dot(b_edge[...], w_msg[...], preferred_element_type=jnp.float32) + b_msg[...]
        bem2_o[...] = jnp.dot(bem, wo_b, preferred_element_type=jnp.float32)


def _edge_kernel(ea, w_edge, b_edge, w3, b_attn, a_edge_o):
    ef = jnp.dot(ea[...], w_edge[...], preferred_element_type=jnp.float32) + b_edge[...]
    a_edge_o[...] = jnp.dot(jnp.tanh(ef), w3[...], preferred_element_type=jnp.float32) + b_attn[...]


def _out_kernel(dx, ag, t, w_out, wem2, bem2, b_out, gamma, beta, out_o):
    w = w_out[...]
    a0s = ag[0]
    a1s = ag[1]
    ts = t[0] + t[1]
    upd = (jnp.dot(dx[...], w[:H, :], preferred_element_type=jnp.float32)
           + jnp.dot(a0s, w[H:H + HH, :], preferred_element_type=jnp.float32)
           + jnp.dot(a1s, w[H + HH:, :], preferred_element_type=jnp.float32)
           + jnp.dot(ts[:, :ED], wem2[...], preferred_element_type=jnp.float32)
           + ts[:, ED:ED + 1] * bem2[...]
           + b_out[...])
    h = dx[...] + upd
    mean = jnp.mean(h, axis=-1, keepdims=True)
    c = h - mean
    var = jnp.mean(c * c, axis=-1, keepdims=True)
    out_o[...] = c * jax.lax.rsqrt(var + 1e-5) * gamma[...] + beta[...]


def _full(shape):
    return pl.BlockSpec(shape, lambda i: (0,) * len(shape))


def _alpha_body(asrc_h, adst_h, aedge_h, sidx_h, didx_h, ea_h,
                alpha_h, t_h,
                asrc_v, adst_v, sidx_v, didx_v, aedge_v, ex_v, denom_v,
                rowbuf_v, rowidx_v, prevcol_v, zbuf_v, eabuf_v, trow_v,
                tidx_v, tidx16_v, dsh, tsh):
    c = lax.axis_index("c")
    s = lax.axis_index("s")
    base1 = pl.multiple_of(s * EPT1, 16)
    zeros = jnp.zeros((L,), jnp.float32)
    iota = lax.iota(jnp.int32, L)

    for k in range(DR // NS):
        zbuf_v[k, :] = zeros
    pltpu.sync_copy(zbuf_v, dsh.at[pl.ds(s * (DR // NS), DR // NS)])
    izeros = jnp.zeros((L,), jnp.int32)

    def zr(j, carry):
        plsc.store_scatter(rowbuf_v, [jnp.full((L,), j, jnp.int32), iota], zeros)
        return carry

    lax.fori_loop(0, RB, zr, None, unroll=False)

    def zp(i, carry):
        plsc.store_scatter(prevcol_v, [i * L + iota], izeros)
        return carry

    lax.fori_loop(0, RB // L, zp, None, unroll=False)
    for j in range(TROWS):
        for k in range(TC_ // L):
            plsc.store_scatter(trow_v, [jnp.full((L,), j, jnp.int32), k * L + iota], zeros)
    for k in range(TR // NS // TROWS):
        pltpu.sync_copy(trow_v, tsh.at[pl.ds(s * (TR // NS) + k * TROWS, TROWS)])
    pltpu.sync_copy(asrc_h, asrc_v)
    pltpu.sync_copy(adst_h, adst_v)
    pltpu.sync_copy(sidx_h.at[pl.ds(base1, EPT1)], sidx_v)
    pltpu.sync_copy(didx_h.at[pl.ds(base1, EPT1)], didx_v)
    pltpu.sync_copy(aedge_h.at[pl.ds(base1, EPT1)], aedge_v)
    plsc.subcore_barrier()

    def group(g, carry):
        o = pl.multiple_of(g * L, 16)
        sidx = sidx_v[pl.ds(o, L)]
        didx = didx_v[pl.ds(o, L)]
        ae = aedge_v[pl.ds(o, L)]
        sa = plsc.load_gather(asrc_v, [sidx])
        sd = plsc.load_gather(adst_v, [didx])
        ex = jnp.exp(sa + sd + ae)
        ex_v[pl.ds(o, L)] = ex
        j = lax.rem(g, RBG)
        rows = j * L + iota
        pc = plsc.load_gather(prevcol_v, [rows])
        plsc.store_scatter(rowbuf_v, [rows, pc], zeros)
        newc = jnp.bitwise_and(didx, L - 1)
        plsc.store_scatter(rowbuf_v, [rows, newc], ex)
        plsc.store_scatter(prevcol_v, [rows], newc)
        plsc.store_scatter(
            rowidx_v,
            [jnp.full((L,), lax.div(j, 5), jnp.int32), lax.rem(j, 5) * L + iota],
            lax.shift_right_logical(didx, 4))

        @pl.when(j == RBG - 1)
        def push():
            for k in range(RBG // 5):
                pltpu.sync_copy(rowbuf_v.at[pl.ds(k * 5 * L, 5 * L)],
                                dsh.at[rowidx_v.at[k]], add=True)
        return carry

    lax.fori_loop(0, G1, group, None, unroll=False)
    plsc.subcore_barrier()

    pltpu.sync_copy(dsh.at[pl.ds(0, DR)], denom_v)
    astart = pl.multiple_of(c * NA0, 16)
    ng = G0 - c

    def agroup(g, carry):
        o = pl.multiple_of(astart + g * L, 16)
        didx = didx_v[pl.ds(o, L)]
        ex = ex_v[pl.ds(o, L)]
        d = plsc.load_gather(
            denom_v, [lax.shift_right_logical(didx, 4), jnp.bitwise_and(didx, L - 1)])
        al = ex / d
        aedge_v[pl.ds(o, L)] = al
        jb = lax.rem(g, TB)

        @pl.when(jb == 0)
        def stage():
            pltpu.sync_copy(ea_h.at[pl.ds(base1 + o, TROWS)], eabuf_v)

        rows16 = jb * L + iota
        for cidx in range(ED):
            pcid = jnp.full((L,), cidx, jnp.int32)
            colv = plsc.load_gather(eabuf_v, [rows16, pcid])
            plsc.store_scatter(trow_v, [rows16, pcid], colv * al)
        plsc.store_scatter(trow_v, [rows16, jnp.full((L,), ED, jnp.int32)], al)
        plsc.store_scatter(tidx_v, [jb * L + iota], didx)
        plsc.store_scatter(tidx16_v, [iota], didx)

        @pl.when(jb == TB - 1)
        def pusht():
            pltpu.sync_copy(trow_v, tsh.at[tidx_v], add=True)

        @pl.when(jnp.logical_and(g == ng - 1, jb != TB - 1))
        def pushlast():
            pltpu.sync_copy(trow_v.at[pl.ds(0, L)], tsh.at[tidx16_v], add=True)
        return carry

    lax.fori_loop(0, ng, agroup, None, unroll=False)

    @pl.when(c == 0)
    def outa0():
        pltpu.sync_copy(aedge_v.at[pl.ds(0, NA0)], alpha_h.at[pl.ds(base1, NA0)])

    @pl.when(c == 1)
    def outa1():
        pltpu.sync_copy(aedge_v.at[pl.ds(NA0, NA1)], alpha_h.at[pl.ds(base1 + NA0, NA1)])

    plsc.subcore_barrier()
    pltpu.sync_copy(tsh.at[pl.ds(s * (TR // NS), TR // NS)],
                    t_h.at[c].at[pl.ds(s * (TR // NS), TR // NS)])


def _spl(x):
    return jnp.full((L,), x, jnp.int32)


def _agg_body(m_h, alpha_h, sidx_h, didx_h, out_h,
              mbuf, sidxc, didxc, alphac, agg_sh,
              sem_s, sem_d, sem_a, sem_g, sem_p):
    c = lax.axis_index("c")
    s = lax.axis_index("s")
    eb = pl.multiple_of(s * EPT2, 8)
    iota = lax.iota(jnp.int32, L)
    zeros = jnp.zeros((L,), jnp.float32)
    moff = c * N_NODES

    def zrow(j, carry):
        for k in range(HH // L):
            plsc.store_scatter(mbuf, [_spl(0), _spl(j), k * L + iota], zeros)
        return carry

    lax.fori_loop(0, BCH, zrow, None, unroll=False)
    for t in range(RPT // BCH):
        pltpu.sync_copy(mbuf.at[0], agg_sh.at[pl.ds(s * RPT + t * BCH, BCH)])
    _rem = RPT % BCH
    if _rem:
        pltpu.sync_copy(mbuf.at[0].at[pl.ds(0, _rem)],
                        agg_sh.at[pl.ds(s * RPT + RPT - _rem, _rem)])
    plsc.subcore_barrier()

    def stage(x):
        o = pl.multiple_of(eb + x * BCH, 8)
        sl = lax.rem(x, NBI)
        pltpu.async_copy(sidx_h.at[pl.ds(o, BCH)], sidxc.at[sl], sem_s)
        pltpu.async_copy(didx_h.at[pl.ds(o, BCH)], didxc.at[sl], sem_d)
        pltpu.async_copy(alpha_h.at[pl.ds(o, BCH)], alphac.at[sl], sem_a)

    def wait_stage(x):
        o = pl.multiple_of(eb + x * BCH, 8)
        sl = lax.rem(x, NBI)
        pltpu.make_async_copy(sidx_h.at[pl.ds(o, BCH)], sidxc.at[sl], sem_s).wait()
        pltpu.make_async_copy(didx_h.at[pl.ds(o, BCH)], didxc.at[sl], sem_d).wait()
        pltpu.make_async_copy(alpha_h.at[pl.ds(o, BCH)], alphac.at[sl], sem_a).wait()
        for k in range(BCH // L):
            v = plsc.load_gather(sidxc, [_spl(sl), k * L + iota])
            plsc.store_scatter(sidxc, [_spl(sl), k * L + iota], v + moff)

    def start_gather(x):
        pltpu.async_copy(m_h.at[sidxc.at[lax.rem(x, NBI)]],
                         mbuf.at[lax.rem(x, NBM)], sem_g)

    def wait_gather(x):
        pltpu.make_async_copy(m_h.at[sidxc.at[lax.rem(x, NBI)]],
                              mbuf.at[lax.rem(x, NBM)], sem_g).wait()

    def push(x):
        pltpu.async_copy(mbuf.at[lax.rem(x, NBM)],
                         agg_sh.at[didxc.at[lax.rem(x, NBI)]], sem_p, add=True)

    def drain_push(x):
        pltpu.make_async_copy(mbuf.at[lax.rem(x, NBM)],
                              agg_sh.at[didxc.at[lax.rem(x, NBI)]], sem_p).wait()

    stage(0)
    stage(1)
    wait_stage(0)
    start_gather(0)

    def chunk(ch, carry):
        @pl.when(ch >= 2)
        def dr():
            drain_push(ch - 2)

        @pl.when(ch + 1 < NCH)
        def ws():
            wait_stage(ch + 1)

        wait_gather(ch)

        @pl.when(ch + 1 < NCH)
        def sg():
            start_gather(ch + 1)

        @pl.when(ch + 2 < NCH)
        def st():
            stage(ch + 2)

        slm = lax.rem(ch, NBM)
        sl = lax.rem(ch, NBI)

        pslm = _spl(slm)
        psl = _spl(sl)

        def edge(j4, ecarry):
            for u in range(4):
                pj = _spl(j4 * 4 + u)
                ab = plsc.load_gather(alphac, [psl, pj])
                for k in range(HH // L):
                    m = plsc.load_gather(mbuf, [pslm, pj, k * L + iota])
                    plsc.store_scatter(mbuf, [pslm, pj, k * L + iota], m * ab)
            return ecarry

        lax.fori_loop(0, BCH // 4, edge, None, unroll=False)
        push(ch)
        return carry

    lax.fori_loop(0, NCH, chunk, None, unroll=False)
    drain_push(NCH - 2)
    drain_push(NCH - 1)
    plsc.subcore_barrier()
    pltpu.sync_copy(agg_sh.at[pl.ds(s * RPT, RPT)], out_h.at[c].at[pl.ds(s * RPT, RPT)])


_SC_MESH = plsc.VectorSubcoreMesh(core_axis_name="c", subcore_axis_name="s")
_SC_PARAMS = pltpu.CompilerParams(needs_layout_passes=False,
                                  use_tc_tiling_on_sc=False)

_alpha_call = pl.kernel(
    _alpha_body,
    out_type=(jax.ShapeDtypeStruct((N_EDGES,), jnp.float32),
              jax.ShapeDtypeStruct((NC, TR, TC_), jnp.float32)),
    mesh=_SC_MESH,
    compiler_params=_SC_PARAMS,
    scratch_types=[
        pltpu.VMEM((N_NODES,), jnp.float32),
        pltpu.VMEM((N_NODES,), jnp.float32),
        pltpu.VMEM((EPT1,), jnp.int32),
        pltpu.VMEM((EPT1,), jnp.int32),
        pltpu.VMEM((EPT1,), jnp.float32),
        pltpu.VMEM((EPT1,), jnp.float32),
        pltpu.VMEM((DR, L), jnp.float32),
        pltpu.VMEM((RB, L), jnp.float32),
        pltpu.VMEM((RBG // 5, 5 * L), jnp.int32),
        pltpu.VMEM((RB,), jnp.int32),
        pltpu.VMEM((DR // NS, L), jnp.float32),
        pltpu.VMEM((TROWS, ED), jnp.float32),
        pltpu.VMEM((TROWS, TC_), jnp.float32),
        pltpu.VMEM((TROWS,), jnp.int32),
        pltpu.VMEM((L,), jnp.int32),
        pltpu.VMEM_SHARED((DR, L), jnp.float32),
        pltpu.VMEM_SHARED((TR, TC_), jnp.float32),
    ],
)

_agg_call = pl.kernel(
    _agg_body,
    out_type=jax.ShapeDtypeStruct((NC, N_NODES, HH), jnp.float32),
    mesh=_SC_MESH,
    compiler_params=_SC_PARAMS,
    scratch_types=[
        pltpu.VMEM((NBM, BCH, HH), jnp.float32),
        pltpu.VMEM((NBI, BCH), jnp.int32),
        pltpu.VMEM((NBI, BCH), jnp.int32),
        pltpu.VMEM((NBI, BCH), jnp.float32),
        pltpu.VMEM_SHARED((N_NODES, HH), jnp.float32),
        pltpu.SemaphoreType.DMA,
        pltpu.SemaphoreType.DMA,
        pltpu.SemaphoreType.DMA,
        pltpu.SemaphoreType.DMA,
        pltpu.SemaphoreType.DMA,
    ],
)


def kernel(src_x, dst_x, edge_index, edge_attr, W_src, b_src, W_dst, b_dst,
           W_edge, b_edge, W_attn, b_attn, W_msg, b_msg, W_out, b_out,
           gamma, beta):
    src_idx = edge_index[0].astype(jnp.int32)
    dst_idx = edge_index[1].astype(jnp.int32)

    b_src2 = b_src.reshape(1, H)
    b_dst2 = b_dst.reshape(1, H)
    b_edge2 = b_edge.reshape(1, H)
    b_msg2 = b_msg.reshape(1, H)
    b_out2 = b_out.reshape(1, H)
    gamma2 = gamma.reshape(1, H)
    beta2 = beta.reshape(1, H)
    w1 = W_attn[:H]
    w2 = W_attn[H:2 * H]
    w3 = W_attn[2 * H:]
    b_attn2 = b_attn.reshape(1, 1)

    nsteps = N_NODES // BN
    a_src, a_dst, m01, wem2, bem2 = pl.pallas_call(
        _node_kernel,
        grid=(nsteps,),
        in_specs=[
            pl.BlockSpec((BN, H), lambda i: (i, 0)),
            pl.BlockSpec((BN, H), lambda i: (i, 0)),
            _full((H, H)), _full((1, H)), _full((H, H)), _full((1, H)),
            _full((H, H)), _full((H, 1)), _full((H, 1)),
            _full((ED, H)), _full((1, H)), _full((1, H)), _full((2 * H, H)),
        ],
        out_specs=[
            pl.BlockSpec((BN, 1), lambda i: (i, 0)),
            pl.BlockSpec((BN, 1), lambda i: (i, 0)),
            pl.BlockSpec((NC, BN, HH), lambda i: (0, i, 0)),
            _full((ED, H)), _full((1, H)),
        ],
        out_shape=[
            jax.ShapeDtypeStruct((N_NODES, 1), jnp.float32),
            jax.ShapeDtypeStruct((N_NODES, 1), jnp.float32),
            jax.ShapeDtypeStruct((NC, N_NODES, HH), jnp.float32),
            jax.ShapeDtypeStruct((ED, H), jnp.float32),
            jax.ShapeDtypeStruct((1, H), jnp.float32),
        ],
    )(src_x, dst_x, W_src, b_src2, W_dst, b_dst2, W_msg, w1, w2,
      W_edge, b_edge2, b_msg2, W_out)

    esteps = N_EDGES // BE
    a_edge = pl.pallas_call(
        _edge_kernel,
        grid=(esteps,),
        in_specs=[
            pl.BlockSpec((BE, ED), lambda i: (i, 0)),
            _full((ED, H)), _full((1, H)), _full((H, 1)), _full((1, 1)),
        ],
        out_specs=pl.BlockSpec((BE, 1), lambda i: (i, 0)),
        out_shape=jax.ShapeDtypeStruct((N_EDGES, 1), jnp.float32),
    )(edge_attr, W_edge, b_edge2, w3, b_attn2)

    alpha, t_mom = _alpha_call(a_src.reshape(N_NODES), a_dst.reshape(N_NODES),
                               a_edge.reshape(N_EDGES), src_idx, dst_idx,
                               edge_attr)
    agg = _agg_call(m01.reshape(NC * N_NODES, HH), alpha, src_idx, dst_idx)

    out = pl.pallas_call(
        _out_kernel,
        grid=(nsteps,),
        in_specs=[
            pl.BlockSpec((BN, H), lambda i: (i, 0)),
            pl.BlockSpec((NC, BN, HH), lambda i: (0, i, 0)),
            pl.BlockSpec((NC, BN, TC_), lambda i: (0, i, 0)),
            _full((2 * H, H)), _full((ED, H)), _full((1, H)),
            _full((1, H)), _full((1, H)), _full((1, H)),
        ],
        out_specs=pl.BlockSpec((BN, H), lambda i: (i, 0)),
        out_shape=jax.ShapeDtypeStruct((N_NODES, H), jnp.float32),
    )(dst_x, agg, t_mom, W_out, wem2, bem2, b_out2, gamma2, beta2)
    return out


# speedup vs baseline: 1.0591x; 1.0591x over previous
import jax
import jax.numpy as jnp
from jax import lax
from jax.experimental import pallas as pl
from jax.experimental.pallas import tpu as pltpu
from jax.experimental.pallas import tpu_sc as plsc

N_NODES = 10000
N_EDGES = 160000
H = 256
HH = 128
ED = 16

BN = 1000
BE = 2000

NC = 2
NS = 16
L = 16

EPT1 = N_EDGES // NS
G1 = EPT1 // L
RBG = 25
RB = RBG * L
NA0 = 5008
NA1 = EPT1 - NA0
G0 = NA0 // L
DR = 640
TB = 8
TROWS = TB * L
TC_ = ED + L
TR = 10240

NW = NC * NS
EPT2 = N_EDGES // NS
BCH = 80
NCH = EPT2 // BCH
NBM = 3
NBI = 5
RPT = N_NODES // NS


def _node_kernel(src_x, dst_x, w_src, b_src, w_dst, b_dst, w_msg, w1, w2,
                 w_edge, b_edge, b_msg, w_out,
                 a_src_o, a_dst_o, m01_o, wem2_o, bem2_o):
    xs = jnp.dot(src_x[...], w_src[...], preferred_element_type=jnp.float32) + b_src[...]
    a_src_o[...] = jnp.dot(jnp.tanh(xs), w1[...], preferred_element_type=jnp.float32)
    m = jnp.dot(xs, w_msg[...], preferred_element_type=jnp.float32)
    m01_o[0] = m[:, :HH]
    m01_o[1] = m[:, HH:]
    xd = jnp.dot(dst_x[...], w_dst[...], preferred_element_type=jnp.float32) + b_dst[...]
    a_dst_o[...] = jnp.dot(jnp.tanh(xd), w2[...], preferred_element_type=jnp.float32)

    @pl.when(pl.program_id(0) == 0)
    def _():
        wo_b = w_out[H:, :]
        wem = jnp.dot(w_edge[...], w_msg[...], preferred_element_type=jnp.float32)
        wem2_o[...] = jnp.dot(wem, wo_b, preferred_element_type=jnp.float32)
        bem = jnp.dot(b_edge[...], w_msg[...], preferred_element_type=jnp.float32) + b_msg[...]
        bem2_o[...] = jnp.dot(bem, wo_b, preferred_element_type=jnp.float32)


def _edge_kernel(ea, w_edge, b_edge, w3, b_attn, a_edge_o):
    ef = jnp.dot(ea[...], w_edge[...], preferred_element_type=jnp.float32) + b_edge[...]
    a_edge_o[...] = jnp.dot(jnp.tanh(ef), w3[...], preferred_element_type=jnp.float32) + b_attn[...]


def _out_kernel(dx, ag, t, w_out, wem2, bem2, b_out, gamma, beta, out_o):
    w = w_out[...]
    a0s = ag[0]
    a1s = ag[1]
    ts = t[0] + t[1]
    upd = (jnp.dot(dx[...], w[:H, :], preferred_element_type=jnp.float32)
           + jnp.dot(a0s, w[H:H + HH, :], preferred_element_type=jnp.float32)
           + jnp.dot(a1s, w[H + HH:, :], preferred_element_type=jnp.float32)
           + jnp.dot(ts[:, :ED], wem2[...], preferred_element_type=jnp.float32)
           + ts[:, ED:ED + 1] * bem2[...]
           + b_out[...])
    h = dx[...] + upd
    mean = jnp.mean(h, axis=-1, keepdims=True)
    c = h - mean
    var = jnp.mean(c * c, axis=-1, keepdims=True)
    out_o[...] = c * jax.lax.rsqrt(var + 1e-5) * gamma[...] + beta[...]


def _full(shape):
    return pl.BlockSpec(shape, lambda i: (0,) * len(shape))


def _alpha_body(asrc_h, adst_h, aedge_h, sidx_h, didx_h, ea_h,
                alpha_h, t_h,
                asrc_v, adst_v, sidx_v, didx_v, aedge_v, ex_v, denom_v,
                rowbuf_v, rowidx_v, zbuf_v, eabuf_v, trow_v, tidx_v, tidx16_v,
                dsh, tsh):
    c = lax.axis_index("c")
    s = lax.axis_index("s")
    base1 = pl.multiple_of(s * EPT1, 16)
    zeros = jnp.zeros((L,), jnp.float32)
    iota = lax.iota(jnp.int32, L)

    for k in range(DR // NS):
        zbuf_v[k, :] = zeros
    pltpu.sync_copy(zbuf_v, dsh.at[pl.ds(s * (DR // NS), DR // NS)])
    for j in range(TROWS):
        for k in range(TC_ // L):
            plsc.store_scatter(trow_v, [jnp.full((L,), j, jnp.int32), k * L + iota], zeros)
    for k in range(TR // NS // TROWS):
        pltpu.sync_copy(trow_v, tsh.at[pl.ds(s * (TR // NS) + k * TROWS, TROWS)])
    pltpu.sync_copy(asrc_h, asrc_v)
    pltpu.sync_copy(adst_h, adst_v)
    pltpu.sync_copy(sidx_h.at[pl.ds(base1, EPT1)], sidx_v)
    pltpu.sync_copy(didx_h.at[pl.ds(base1, EPT1)], didx_v)
    pltpu.sync_copy(aedge_h.at[pl.ds(base1, EPT1)], aedge_v)
    plsc.subcore_barrier()

    def group(g, carry):
        o = pl.multiple_of(g * L, 16)
        sidx = sidx_v[pl.ds(o, L)]
        didx = didx_v[pl.ds(o, L)]
        ae = aedge_v[pl.ds(o, L)]
        sa = plsc.load_gather(asrc_v, [sidx])
        sd = plsc.load_gather(adst_v, [didx])
        ex = jnp.exp(sa + sd + ae)
        ex_v[pl.ds(o, L)] = ex
        j = lax.rem(g, RBG)
        rows = j * L + iota
        for k in range(L):
            plsc.store_scatter(rowbuf_v, [jnp.full((L,), j * L + k, jnp.int32), iota], zeros)
        plsc.store_scatter(rowbuf_v, [rows, jnp.bitwise_and(didx, L - 1)], ex)
        plsc.store_scatter(
            rowidx_v,
            [jnp.full((L,), lax.div(j, 5), jnp.int32), lax.rem(j, 5) * L + iota],
            lax.shift_right_logical(didx, 4))

        @pl.when(j == RBG - 1)
        def push():
            for k in range(RBG // 5):
                pltpu.sync_copy(rowbuf_v.at[pl.ds(k * 5 * L, 5 * L)],
                                dsh.at[rowidx_v.at[k]], add=True)
        return carry

    lax.fori_loop(0, G1, group, None, unroll=False)
    plsc.subcore_barrier()

    pltpu.sync_copy(dsh.at[pl.ds(0, DR)], denom_v)
    astart = pl.multiple_of(c * NA0, 16)
    ng = G0 - c

    def agroup(g, carry):
        o = pl.multiple_of(astart + g * L, 16)
        didx = didx_v[pl.ds(o, L)]
        ex = ex_v[pl.ds(o, L)]
        d = plsc.load_gather(
            denom_v, [lax.shift_right_logical(didx, 4), jnp.bitwise_and(didx, L - 1)])
        al = ex / d
        aedge_v[pl.ds(o, L)] = al
        jb = lax.rem(g, TB)

        @pl.when(jb == 0)
        def stage():
            pltpu.sync_copy(ea_h.at[pl.ds(base1 + o, TROWS)], eabuf_v)

        def edge(j, ecarry):
            row = jnp.full((L,), jb * L + j, jnp.int32)
            ab = plsc.load_gather(aedge_v, [jnp.full((L,), o + j, jnp.int32)])
            r = plsc.load_gather(eabuf_v, [row, iota])
            plsc.store_scatter(trow_v, [row, iota], r * ab)
            return ecarry

        lax.fori_loop(0, L, edge, None, unroll=False)
        plsc.store_scatter(trow_v, [jb * L + iota, jnp.full((L,), ED, jnp.int32)], al)
        plsc.store_scatter(tidx_v, [jb * L + iota], didx)
        plsc.store_scatter(tidx16_v, [iota], didx)

        @pl.when(jb == TB - 1)
        def pusht():
            pltpu.sync_copy(trow_v, tsh.at[tidx_v], add=True)

        @pl.when(jnp.logical_and(g == ng - 1, jb != TB - 1))
        def pushlast():
            pltpu.sync_copy(trow_v.at[pl.ds(0, L)], tsh.at[tidx16_v], add=True)
        return carry

    lax.fori_loop(0, ng, agroup, None, unroll=False)

    @pl.when(c == 0)
    def outa0():
        pltpu.sync_copy(aedge_v.at[pl.ds(0, NA0)], alpha_h.at[pl.ds(base1, NA0)])

    @pl.when(c == 1)
    def outa1():
        pltpu.sync_copy(aedge_v.at[pl.ds(NA0, NA1)], alpha_h.at[pl.ds(base1 + NA0, NA1)])

    plsc.subcore_barrier()
    pltpu.sync_copy(tsh.at[pl.ds(s * (TR // NS), TR // NS)],
                    t_h.at[c].at[pl.ds(s * (TR // NS), TR // NS)])


def _spl(x):
    return jnp.full((L,), x, jnp.int32)


def _agg_body(m_h, alpha_h, sidx_h, didx_h, out_h,
              mbuf, sidxc, didxc, alphac, agg_sh,
              sem_s, sem_d, sem_a, sem_g, sem_p):
    c = lax.axis_index("c")
    s = lax.axis_index("s")
    eb = pl.multiple_of(s * EPT2, 8)
    iota = lax.iota(jnp.int32, L)
    zeros = jnp.zeros((L,), jnp.float32)
    moff = c * N_NODES

    def zrow(j, carry):
        for k in range(HH // L):
            plsc.store_scatter(mbuf, [_spl(0), _spl(j), k * L + iota], zeros)
        return carry

    lax.fori_loop(0, BCH, zrow, None, unroll=False)
    for t in range(RPT // BCH):
        pltpu.sync_copy(mbuf.at[0], agg_sh.at[pl.ds(s * RPT + t * BCH, BCH)])
    _rem = RPT % BCH
    if _rem:
        pltpu.sync_copy(mbuf.at[0].at[pl.ds(0, _rem)],
                        agg_sh.at[pl.ds(s * RPT + RPT - _rem, _rem)])
    plsc.subcore_barrier()

    def stage(x):
        o = pl.multiple_of(eb + x * BCH, 8)
        sl = lax.rem(x, NBI)
        pltpu.async_copy(sidx_h.at[pl.ds(o, BCH)], sidxc.at[sl], sem_s)
        pltpu.async_copy(didx_h.at[pl.ds(o, BCH)], didxc.at[sl], sem_d)
        pltpu.async_copy(alpha_h.at[pl.ds(o, BCH)], alphac.at[sl], sem_a)

    def wait_stage(x):
        o = pl.multiple_of(eb + x * BCH, 8)
        sl = lax.rem(x, NBI)
        pltpu.make_async_copy(sidx_h.at[pl.ds(o, BCH)], sidxc.at[sl], sem_s).wait()
        pltpu.make_async_copy(didx_h.at[pl.ds(o, BCH)], didxc.at[sl], sem_d).wait()
        pltpu.make_async_copy(alpha_h.at[pl.ds(o, BCH)], alphac.at[sl], sem_a).wait()
        for k in range(BCH // L):
            v = plsc.load_gather(sidxc, [_spl(sl), k * L + iota])
            plsc.store_scatter(sidxc, [_spl(sl), k * L + iota], v + moff)

    def start_gather(x):
        pltpu.async_copy(m_h.at[sidxc.at[lax.rem(x, NBI)]],
                         mbuf.at[lax.rem(x, NBM)], sem_g)

    def wait_gather(x):
        pltpu.make_async_copy(m_h.at[sidxc.at[lax.rem(x, NBI)]],
                              mbuf.at[lax.rem(x, NBM)], sem_g).wait()

    def push(x):
        pltpu.async_copy(mbuf.at[lax.rem(x, NBM)],
                         agg_sh.at[didxc.at[lax.rem(x, NBI)]], sem_p, add=True)

    def drain_push(x):
        pltpu.make_async_copy(mbuf.at[lax.rem(x, NBM)],
                              agg_sh.at[didxc.at[lax.rem(x, NBI)]], sem_p).wait()

    stage(0)
    stage(1)
    wait_stage(0)
    start_gather(0)

    def chunk(ch, carry):
        @pl.when(ch >= 2)
        def dr():
            drain_push(ch - 2)

        @pl.when(ch + 1 < NCH)
        def ws():
            wait_stage(ch + 1)

        wait_gather(ch)

        @pl.when(ch + 1 < NCH)
        def sg():
            start_gather(ch + 1)

        @pl.when(ch + 2 < NCH)
        def st():
            stage(ch + 2)

        slm = lax.rem(ch, NBM)
        sl = lax.rem(ch, NBI)

        pslm = _spl(slm)
        psl = _spl(sl)

        def edge(j4, ecarry):
            for u in range(4):
                pj = _spl(j4 * 4 + u)
                ab = plsc.load_gather(alphac, [psl, pj])
                for k in range(HH // L):
                    m = plsc.load_gather(mbuf, [pslm, pj, k * L + iota])
                    plsc.store_scatter(mbuf, [pslm, pj, k * L + iota], m * ab)
            return ecarry

        lax.fori_loop(0, BCH // 4, edge, None, unroll=False)
        push(ch)
        return carry

    lax.fori_loop(0, NCH, chunk, None, unroll=False)
    drain_push(NCH - 2)
    drain_push(NCH - 1)
    plsc.subcore_barrier()
    pltpu.sync_copy(agg_sh.at[pl.ds(s * RPT, RPT)], out_h.at[c].at[pl.ds(s * RPT, RPT)])


_SC_MESH = plsc.VectorSubcoreMesh(core_axis_name="c", subcore_axis_name="s")
_SC_PARAMS = pltpu.CompilerParams(needs_layout_passes=False,
                                  use_tc_tiling_on_sc=False)

_alpha_call = pl.kernel(
    _alpha_body,
    out_type=(jax.ShapeDtypeStruct((N_EDGES,), jnp.float32),
              jax.ShapeDtypeStruct((NC, TR, TC_), jnp.float32)),
    mesh=_SC_MESH,
    compiler_params=_SC_PARAMS,
    scratch_types=[
        pltpu.VMEM((N_NODES,), jnp.float32),
        pltpu.VMEM((N_NODES,), jnp.float32),
        pltpu.VMEM((EPT1,), jnp.int32),
        pltpu.VMEM((EPT1,), jnp.int32),
        pltpu.VMEM((EPT1,), jnp.float32),
        pltpu.VMEM((EPT1,), jnp.float32),
        pltpu.VMEM((DR, L), jnp.float32),
        pltpu.VMEM((RB, L), jnp.float32),
        pltpu.VMEM((RBG // 5, 5 * L), jnp.int32),
        pltpu.VMEM((DR // NS, L), jnp.float32),
        pltpu.VMEM((TROWS, ED), jnp.float32),
        pltpu.VMEM((TROWS, TC_), jnp.float32),
        pltpu.VMEM((TROWS,), jnp.int32),
        pltpu.VMEM((L,), jnp.int32),
        pltpu.VMEM_SHARED((DR, L), jnp.float32),
        pltpu.VMEM_SHARED((TR, TC_), jnp.float32),
    ],
)

_agg_call = pl.kernel(
    _agg_body,
    out_type=jax.ShapeDtypeStruct((NC, N_NODES, HH), jnp.float32),
    mesh=_SC_MESH,
    compiler_params=_SC_PARAMS,
    scratch_types=[
        pltpu.VMEM((NBM, BCH, HH), jnp.float32),
        pltpu.VMEM((NBI, BCH), jnp.int32),
        pltpu.VMEM((NBI, BCH), jnp.int32),
        pltpu.VMEM((NBI, BCH), jnp.float32),
        pltpu.VMEM_SHARED((N_NODES, HH), jnp.float32),
        pltpu.SemaphoreType.DMA,
        pltpu.SemaphoreType.DMA,
        pltpu.SemaphoreType.DMA,
        pltpu.SemaphoreType.DMA,
        pltpu.SemaphoreType.DMA,
    ],
)


def kernel(src_x, dst_x, edge_index, edge_attr, W_src, b_src, W_dst, b_dst,
           W_edge, b_edge, W_attn, b_attn, W_msg, b_msg, W_out, b_out,
           gamma, beta):
    src_idx = edge_index[0].astype(jnp.int32)
    dst_idx = edge_index[1].astype(jnp.int32)

    b_src2 = b_src.reshape(1, H)
    b_dst2 = b_dst.reshape(1, H)
    b_edge2 = b_edge.reshape(1, H)
    b_msg2 = b_msg.reshape(1, H)
    b_out2 = b_out.reshape(1, H)
    gamma2 = gamma.reshape(1, H)
    beta2 = beta.reshape(1, H)
    w1 = W_attn[:H]
    w2 = W_attn[H:2 * H]
    w3 = W_attn[2 * H:]
    b_attn2 = b_attn.reshape(1, 1)

    nsteps = N_NODES // BN
    a_src, a_dst, m01, wem2, bem2 = pl.pallas_call(
        _node_kernel,
        grid=(nsteps,),
        in_specs=[
            pl.BlockSpec((BN, H), lambda i: (i, 0)),
            pl.BlockSpec((BN, H), lambda i: (i, 0)),
            _full((H, H)), _full((1, H)), _full((H, H)), _full((1, H)),
            _full((H, H)), _full((H, 1)), _full((H, 1)),
            _full((ED, H)), _full((1, H)), _full((1, H)), _full((2 * H, H)),
        ],
        out_specs=[
            pl.BlockSpec((BN, 1), lambda i: (i, 0)),
            pl.BlockSpec((BN, 1), lambda i: (i, 0)),
            pl.BlockSpec((NC, BN, HH), lambda i: (0, i, 0)),
            _full((ED, H)), _full((1, H)),
        ],
        out_shape=[
            jax.ShapeDtypeStruct((N_NODES, 1), jnp.float32),
            jax.ShapeDtypeStruct((N_NODES, 1), jnp.float32),
            jax.ShapeDtypeStruct((NC, N_NODES, HH), jnp.float32),
            jax.ShapeDtypeStruct((ED, H), jnp.float32),
            jax.ShapeDtypeStruct((1, H), jnp.float32),
        ],
    )(src_x, dst_x, W_src, b_src2, W_dst, b_dst2, W_msg, w1, w2,
      W_edge, b_edge2, b_msg2, W_out)

    esteps = N_EDGES // BE
    a_edge = pl.pallas_call(
        _edge_kernel,
        grid=(esteps,),
        in_specs=[
            pl.BlockSpec((BE, ED), lambda i: (i, 0)),
            _full((ED, H)), _full((1, H)), _full((H, 1)), _full((1, 1)),
        ],
        out_specs=pl.BlockSpec((BE, 1), lambda i: (i, 0)),
        out_shape=jax.ShapeDtypeStruct((N_EDGES, 1), jnp.float32),
    )(edge_attr, W_edge, b_edge2, w3, b_attn2)

    alpha, t_mom = _alpha_call(a_src.reshape(N_NODES), a_dst.reshape(N_NODES),
                               a_edge.reshape(N_EDGES), src_idx, dst_idx,
                               edge_attr)
    agg = _agg_call(m01.reshape(NC * N_NODES, HH), alpha, src_idx, dst_idx)

    out = pl.pallas_call(
        _out_kernel,
        grid=(nsteps,),
        in_specs=[
            pl.BlockSpec((BN, H), lambda i: (i, 0)),
            pl.BlockSpec((NC, BN, HH), lambda i: (0, i, 0)),
            pl.BlockSpec((NC, BN, TC_), lambda i: (0, i, 0)),
            _full((2 * H, H)), _full((ED, H)), _full((1, H)),
            _full((1, H)), _full((1, H)), _full((1, H)),
        ],
        out_specs=pl.BlockSpec((BN, H), lambda i: (i, 0)),
        out_shape=jax.ShapeDtypeStruct((N_NODES, H), jnp.float32),
    )(dst_x, agg, t_mom, W_out, wem2, bem2, b_out2, gamma2, beta2)
    return out
